# baseline (speedup 1.0000x reference)
import jax
import jax.numpy as jnp
from jax import lax
from jax.experimental import pallas as pl
from jax.experimental.pallas import tpu as pltpu
from jax.experimental.pallas import tpu_sc as plsc

N = 10000
D = 128
E = 320000
K = 80
NSUB = 16
CPT = E // NSUB // K
SB = 10
NB = CPT // SB
RPT = 624


def _spmm_body(rows_hbm, cols_hbm, vals_hbm, x_hbm, out_hbm,
               rows_v, cols_v, vals_v, gbuf, sbuf, acc, gsem, ssem):
    c = lax.axis_index("c")
    s = lax.axis_index("s")

    def _gather(i, b):
        return pltpu.make_async_copy(x_hbm.at[cols_v.at[i]], gbuf.at[b],
                                     gsem.at[b])

    def _scatter(i, b):
        return pltpu.make_async_copy(sbuf.at[b], acc.at[rows_v.at[i]],
                                     ssem.at[b])

    def _scale(i, b):
        for g2 in range(K // 16):
            vvec = vals_v[i, pl.ds(16 * g2, 16)]
            for e16 in range(16):
                e = 16 * g2 + e16
                vv = jnp.full((16,), vvec[e16], jnp.float32)
                for j in range(D // 16):
                    sl = pl.ds(16 * j, 16)
                    sbuf[b, e, sl] = gbuf[b, e, sl] * vv

    def _zero_row(r, _):
        for j in range(D // 16):
            gbuf[0, r, pl.ds(16 * j, 16)] = jnp.zeros((16,), jnp.float32)
        return 0
    lax.fori_loop(0, K, _zero_row, 0)
    for k in range(7):
        pltpu.sync_copy(gbuf.at[0], acc.at[pl.ds(s * RPT + 80 * k, 80)])
    pltpu.sync_copy(gbuf.at[0, pl.ds(0, 64)], acc.at[pl.ds(s * RPT + 560, 64)])

    @pl.when(s == NSUB - 1)
    def _():
        pltpu.sync_copy(gbuf.at[0, pl.ds(0, 16)], acc.at[pl.ds(NSUB * RPT, 16)])
    plsc.subcore_barrier()

    def _block(ob, _):
        @pl.when(ob > 0)
        def _():
            _scatter(SB - 2, 0).wait()
            _scatter(SB - 1, 1).wait()

        pltpu.sync_copy(rows_hbm.at[c, s, ob], rows_v)
        pltpu.sync_copy(cols_hbm.at[c, s, ob], cols_v)
        pltpu.sync_copy(vals_hbm.at[c, s, ob], vals_v)
        for b in range(2):
            _gather(b, b).start()

        def _pair(g, _):
            for b in range(2):
                i = 2 * g + b
                _gather(i, b).wait()

                @pl.when(g > 0)
                def _():
                    _scatter(i - 2, b).wait()

                _scale(i, b)
                _scatter(i, b).start(add=True)

                @pl.when(i + 2 < SB)
                def _():
                    _gather(i + 2, b).start()
            return 0
        lax.fori_loop(0, SB // 2, _pair, 0)
        return 0
    lax.fori_loop(0, NB, _block, 0)
    _scatter(SB - 2, 0).wait()
    _scatter(SB - 1, 1).wait()
    plsc.subcore_barrier()

    for k in range(7):
        r0 = s * RPT + 80 * k
        pltpu.sync_copy(acc.at[pl.ds(r0, 80)], gbuf.at[0])
        pltpu.sync_copy(gbuf.at[0], out_hbm.at[c, pl.ds(r0, 80)])
    r0 = s * RPT + 560
    pltpu.sync_copy(acc.at[pl.ds(r0, 64)], gbuf.at[0, pl.ds(0, 64)])
    pltpu.sync_copy(gbuf.at[0, pl.ds(0, 64)], out_hbm.at[c, pl.ds(r0, 64)])

    @pl.when(s == NSUB - 1)
    def _():
        pltpu.sync_copy(acc.at[pl.ds(NSUB * RPT, 16)], gbuf.at[0, pl.ds(0, 16)])
        pltpu.sync_copy(gbuf.at[0, pl.ds(0, 16)],
                        out_hbm.at[c, pl.ds(NSUB * RPT, 16)])


def _make_spmm():
    mesh = plsc.VectorSubcoreMesh(core_axis_name="c", subcore_axis_name="s")
    return pl.kernel(
        _spmm_body,
        out_type=jax.ShapeDtypeStruct((2, N, D), jnp.float32),
        mesh=mesh,
        scratch_types=[
            pltpu.VMEM((SB, K), jnp.int32),
            pltpu.VMEM((SB, K), jnp.int32),
            pltpu.VMEM((SB, K), jnp.float32),
            pltpu.VMEM((2, K, D), jnp.float32),
            pltpu.VMEM((2, K, D), jnp.float32),
            pltpu.VMEM_SHARED((N, D), jnp.float32),
            pltpu.SemaphoreType.DMA((2,)),
            pltpu.SemaphoreType.DMA((2,)),
        ],
    )


def _dense_body(x_ref, w_ref, b_ref, o_ref):
    x = x_ref[0]
    w = w_ref[0]
    b = b_ref[0]
    h = lax.dot_general(x, w, (((1,), (1,)), ((), ())),
                        precision=lax.Precision.HIGHEST,
                        preferred_element_type=jnp.float32)
    h = h + b
    h = jnp.where(h >= 0, h, 0.01 * h)
    n = jnp.sqrt(jnp.sum(h * h, axis=1, keepdims=True))
    o_ref[0] = h / jnp.maximum(n, 1e-12)


BL = 2000


def _dense(x, w, b):
    return pl.pallas_call(
        _dense_body,
        grid=(2, N // BL),
        in_specs=[
            pl.BlockSpec((1, BL, D), lambda g, i: (g, i, 0)),
            pl.BlockSpec((1, D, D), lambda g, i: (g, 0, 0)),
            pl.BlockSpec((1, 1, D), lambda g, i: (g, 0, 0)),
        ],
        out_specs=pl.BlockSpec((1, BL, D), lambda g, i: (g, i, 0)),
        out_shape=jax.ShapeDtypeStruct((2, N, D), jnp.float32),
    )(x, w, b)


def kernel(user_adj_indices, user_adj_values, item_adj_indices, item_adj_values,
           emb_user, emb_item,
           u_W0, u_b0, u_W1, u_b1, i_W0, i_b0, i_W1, i_b1):
    spmm = _make_spmm()

    rows = jnp.stack([user_adj_indices[0], item_adj_indices[0]]) \
        .reshape(2, NSUB, NB, SB, K)
    cols = jnp.stack([user_adj_indices[1], item_adj_indices[1] + N]) \
        .reshape(2, NSUB, NB, SB, K)
    vals = jnp.stack([user_adj_values, item_adj_values]) \
        .reshape(2, NSUB, NB, SB, K)

    w0 = jnp.stack([u_W0, i_W0])
    b0 = jnp.stack([u_b0, i_b0]).reshape(2, 1, D)
    w1 = jnp.stack([u_W1, i_W1])
    b1 = jnp.stack([u_b1, i_b1]).reshape(2, 1, D)

    x = jnp.stack([emb_user, emb_item]).reshape(2 * N, D)
    p = spmm(rows, cols, vals, x)
    x = _dense(p, w0, b0).reshape(2 * N, D)
    p = spmm(rows, cols, vals, x)
    x = _dense(p, w1, b1)
    return (x[0], x[1])


# speedup vs baseline: 1.0492x; 1.0492x over previous
import jax
import jax.numpy as jnp
from jax import lax
from jax.experimental import pallas as pl
from jax.experimental.pallas import tpu as pltpu
from jax.experimental.pallas import tpu_sc as plsc

N = 10000
D = 128
E = 320000
K = 80
NSUB = 16
CPT = E // NSUB // K
SB = 50
NB = CPT // SB
RPT = 624


def _spmm_body(rows_hbm, cols_hbm, vals_hbm, x_hbm, out_hbm,
               rows_v, cols_v, vals_v, gbuf, acc, gsem):
    c = lax.axis_index("c")
    s = lax.axis_index("s")

    def _gather(i, b):
        return pltpu.make_async_copy(x_hbm.at[cols_v.at[i]], gbuf.at[b],
                                     gsem.at[b])

    def _scale(i, b):
        for g2 in range(K // 16):
            vvec = vals_v[i, pl.ds(16 * g2, 16)]
            for e16 in range(16):
                e = 16 * g2 + e16
                vv = jnp.full((16,), vvec[e16], jnp.float32)
                for j in range(D // 16):
                    sl = pl.ds(16 * j, 16)
                    gbuf[b, e, sl] = gbuf[b, e, sl] * vv

    def _zero_row(r, _):
        for j in range(D // 16):
            gbuf[0, r, pl.ds(16 * j, 16)] = jnp.zeros((16,), jnp.float32)
        return 0
    lax.fori_loop(0, K, _zero_row, 0)
    for k in range(7):
        pltpu.sync_copy(gbuf.at[0], acc.at[pl.ds(s * RPT + 80 * k, 80)])
    pltpu.sync_copy(gbuf.at[0, pl.ds(0, 64)], acc.at[pl.ds(s * RPT + 560, 64)])

    @pl.when(s == NSUB - 1)
    def _():
        pltpu.sync_copy(gbuf.at[0, pl.ds(0, 16)], acc.at[pl.ds(NSUB * RPT, 16)])
    plsc.subcore_barrier()

    def _block(ob, _):
        pltpu.sync_copy(rows_hbm.at[c, s, ob], rows_v)
        pltpu.sync_copy(cols_hbm.at[c, s, ob], cols_v)
        pltpu.sync_copy(vals_hbm.at[c, s, ob], vals_v)
        for b in range(2):
            _gather(b, b).start()

        def _pair(g, _):
            for b in range(2):
                i = 2 * g + b
                _gather(i, b).wait()
                _scale(i, b)
                pltpu.sync_copy(gbuf.at[b], acc.at[rows_v.at[i]], add=True)

                @pl.when(i + 2 < SB)
                def _():
                    _gather(i + 2, b).start()
            return 0
        lax.fori_loop(0, SB // 2, _pair, 0)
        return 0
    lax.fori_loop(0, NB, _block, 0)
    plsc.subcore_barrier()

    for k in range(7):
        r0 = s * RPT + 80 * k
        pltpu.sync_copy(acc.at[pl.ds(r0, 80)], gbuf.at[0])
        pltpu.sync_copy(gbuf.at[0], out_hbm.at[c, pl.ds(r0, 80)])
    r0 = s * RPT + 560
    pltpu.sync_copy(acc.at[pl.ds(r0, 64)], gbuf.at[0, pl.ds(0, 64)])
    pltpu.sync_copy(gbuf.at[0, pl.ds(0, 64)], out_hbm.at[c, pl.ds(r0, 64)])

    @pl.when(s == NSUB - 1)
    def _():
        pltpu.sync_copy(acc.at[pl.ds(NSUB * RPT, 16)], gbuf.at[0, pl.ds(0, 16)])
        pltpu.sync_copy(gbuf.at[0, pl.ds(0, 16)],
                        out_hbm.at[c, pl.ds(NSUB * RPT, 16)])


def _make_spmm():
    mesh = plsc.VectorSubcoreMesh(core_axis_name="c", subcore_axis_name="s")
    return pl.kernel(
        _spmm_body,
        out_type=jax.ShapeDtypeStruct((2, N, D), jnp.float32),
        mesh=mesh,
        scratch_types=[
            pltpu.VMEM((SB, K), jnp.int32),
            pltpu.VMEM((SB, K), jnp.int32),
            pltpu.VMEM((SB, K), jnp.float32),
            pltpu.VMEM((2, K, D), jnp.float32),
            pltpu.VMEM_SHARED((N, D), jnp.float32),
            pltpu.SemaphoreType.DMA((2,)),
        ],
    )


def _dense_body(x_ref, w_ref, b_ref, o_ref):
    x = x_ref[0]
    w = w_ref[0]
    b = b_ref[0]
    h = lax.dot_general(x, w, (((1,), (1,)), ((), ())),
                        precision=lax.Precision.HIGHEST,
                        preferred_element_type=jnp.float32)
    h = h + b
    h = jnp.where(h >= 0, h, 0.01 * h)
    n = jnp.sqrt(jnp.sum(h * h, axis=1, keepdims=True))
    o_ref[0] = h / jnp.maximum(n, 1e-12)


BL = 2000


def _dense(x, w, b):
    return pl.pallas_call(
        _dense_body,
        grid=(2, N // BL),
        in_specs=[
            pl.BlockSpec((1, BL, D), lambda g, i: (g, i, 0)),
            pl.BlockSpec((1, D, D), lambda g, i: (g, 0, 0)),
            pl.BlockSpec((1, 1, D), lambda g, i: (g, 0, 0)),
        ],
        out_specs=pl.BlockSpec((1, BL, D), lambda g, i: (g, i, 0)),
        out_shape=jax.ShapeDtypeStruct((2, N, D), jnp.float32),
    )(x, w, b)


def kernel(user_adj_indices, user_adj_values, item_adj_indices, item_adj_values,
           emb_user, emb_item,
           u_W0, u_b0, u_W1, u_b1, i_W0, i_b0, i_W1, i_b1):
    spmm = _make_spmm()

    rows = jnp.stack([user_adj_indices[0], item_adj_indices[0]]) \
        .reshape(2, NSUB, NB, SB, K)
    cols = jnp.stack([user_adj_indices[1], item_adj_indices[1] + N]) \
        .reshape(2, NSUB, NB, SB, K)
    vals = jnp.stack([user_adj_values, item_adj_values]) \
        .reshape(2, NSUB, NB, SB, K)

    w0 = jnp.stack([u_W0, i_W0])
    b0 = jnp.stack([u_b0, i_b0]).reshape(2, 1, D)
    w1 = jnp.stack([u_W1, i_W1])
    b1 = jnp.stack([u_b1, i_b1]).reshape(2, 1, D)

    x = jnp.stack([emb_user, emb_item]).reshape(2 * N, D)
    p = spmm(rows, cols, vals, x)
    x = _dense(p, w0, b0).reshape(2 * N, D)
    p = spmm(rows, cols, vals, x)
    x = _dense(p, w1, b1)
    return (x[0], x[1])


# speedup vs baseline: 1.0639x; 1.0140x over previous
import jax
import jax.numpy as jnp
from jax import lax
from jax.experimental import pallas as pl
from jax.experimental.pallas import tpu as pltpu
from jax.experimental.pallas import tpu_sc as plsc

N = 10000
D = 128
E = 320000
K = 80
NSUB = 16
CPT = E // NSUB // K
SB = 40
NBM = 6
TB = CPT - NBM * SB
RPT = 624


def _spmm_body(rows_hbm, cols_hbm, vals_hbm, x_hbm, out_hbm,
               rows_v, cols_v, vals_v, gbuf, sbuf, acc, gsem, ssem):
    c = lax.axis_index("c")
    s = lax.axis_index("s")

    def _gather(i, b):
        return pltpu.make_async_copy(x_hbm.at[cols_v.at[i]], gbuf.at[b],
                                     gsem.at[b])

    def _scatter(i):
        return pltpu.make_async_copy(sbuf, acc.at[rows_v.at[i]], ssem)

    def _scale(i, b):
        for g2 in range(K // 16):
            vvec = vals_v[i, pl.ds(16 * g2, 16)]
            for e16 in range(16):
                e = 16 * g2 + e16
                vv = jnp.full((16,), vvec[e16], jnp.float32)
                for j in range(D // 16):
                    sl = pl.ds(16 * j, 16)
                    sbuf[e, sl] = gbuf[b, e, sl] * vv

    def _zero_row(r, _):
        for j in range(D // 16):
            gbuf[0, r, pl.ds(16 * j, 16)] = jnp.zeros((16,), jnp.float32)
        return 0
    lax.fori_loop(0, K, _zero_row, 0)
    for k in range(7):
        pltpu.sync_copy(gbuf.at[0], acc.at[pl.ds(s * RPT + 80 * k, 80)])
    pltpu.sync_copy(gbuf.at[0, pl.ds(0, 64)], acc.at[pl.ds(s * RPT + 560, 64)])

    @pl.when(s == NSUB - 1)
    def _():
        pltpu.sync_copy(gbuf.at[0, pl.ds(0, 16)], acc.at[pl.ds(NSUB * RPT, 16)])
    plsc.subcore_barrier()

    def _run_block(c0, nch, first):
        if first:
            pass
        else:
            _scatter(0).wait()
        pltpu.sync_copy(rows_hbm.at[c, s, pl.ds(c0, nch)],
                        rows_v.at[pl.ds(0, nch)])
        pltpu.sync_copy(cols_hbm.at[c, s, pl.ds(c0, nch)],
                        cols_v.at[pl.ds(0, nch)])
        pltpu.sync_copy(vals_hbm.at[c, s, pl.ds(c0, nch)],
                        vals_v.at[pl.ds(0, nch)])
        for b in range(2):
            _gather(b, b).start()

        def _pair(g, _):
            for b in range(2):
                i = 2 * g + b
                _gather(i, b).wait()

                if b == 0:
                    if first:
                        @pl.when(g > 0)
                        def _():
                            _scatter(i - 1).wait()
                    else:
                        @pl.when(g > 0)
                        def _():
                            _scatter(i - 1).wait()
                else:
                    _scatter(i - 1).wait()

                _scale(i, b)
                _scatter(i).start(add=True)

                @pl.when(i + 2 < nch)
                def _():
                    _gather(i + 2, b).start()
            return 0
        lax.fori_loop(0, nch // 2, _pair, 0)

    def _block(ob, _):
        c0 = pl.multiple_of(ob * SB, SB)
        _run_block(c0, SB, False)
        return 0

    _run_block(0, SB, True)
    lax.fori_loop(1, NBM, _block, 0)
    _run_block(NBM * SB, TB, False)
    _scatter(TB - 1).wait()
    plsc.subcore_barrier()

    for k in range(7):
        r0 = s * RPT + 80 * k
        pltpu.sync_copy(acc.at[pl.ds(r0, 80)], gbuf.at[0])
        pltpu.sync_copy(gbuf.at[0], out_hbm.at[c, pl.ds(r0, 80)])
    r0 = s * RPT + 560
    pltpu.sync_copy(acc.at[pl.ds(r0, 64)], gbuf.at[0, pl.ds(0, 64)])
    pltpu.sync_copy(gbuf.at[0, pl.ds(0, 64)], out_hbm.at[c, pl.ds(r0, 64)])

    @pl.when(s == NSUB - 1)
    def _():
        pltpu.sync_copy(acc.at[pl.ds(NSUB * RPT, 16)], gbuf.at[0, pl.ds(0, 16)])
        pltpu.sync_copy(gbuf.at[0, pl.ds(0, 16)],
                        out_hbm.at[c, pl.ds(NSUB * RPT, 16)])


def _make_spmm():
    mesh = plsc.VectorSubcoreMesh(core_axis_name="c", subcore_axis_name="s")
    return pl.kernel(
        _spmm_body,
        out_type=jax.ShapeDtypeStruct((2, N, D), jnp.float32),
        mesh=mesh,
        scratch_types=[
            pltpu.VMEM((SB, K), jnp.int32),
            pltpu.VMEM((SB, K), jnp.int32),
            pltpu.VMEM((SB, K), jnp.float32),
            pltpu.VMEM((2, K, D), jnp.float32),
            pltpu.VMEM((K, D), jnp.float32),
            pltpu.VMEM_SHARED((N, D), jnp.float32),
            pltpu.SemaphoreType.DMA((2,)),
            pltpu.SemaphoreType.DMA,
        ],
    )


def _dense_body(x_ref, w_ref, b_ref, o_ref):
    x = x_ref[0]
    w = w_ref[0]
    b = b_ref[0]
    h = lax.dot_general(x, w, (((1,), (1,)), ((), ())),
                        precision=lax.Precision.HIGHEST,
                        preferred_element_type=jnp.float32)
    h = h + b
    h = jnp.where(h >= 0, h, 0.01 * h)
    n = jnp.sqrt(jnp.sum(h * h, axis=1, keepdims=True))
    o_ref[0] = h / jnp.maximum(n, 1e-12)


BL = 2000


def _dense(x, w, b):
    return pl.pallas_call(
        _dense_body,
        grid=(2, N // BL),
        in_specs=[
            pl.BlockSpec((1, BL, D), lambda g, i: (g, i, 0)),
            pl.BlockSpec((1, D, D), lambda g, i: (g, 0, 0)),
            pl.BlockSpec((1, 1, D), lambda g, i: (g, 0, 0)),
        ],
        out_specs=pl.BlockSpec((1, BL, D), lambda g, i: (g, i, 0)),
        out_shape=jax.ShapeDtypeStruct((2, N, D), jnp.float32),
    )(x, w, b)


def kernel(user_adj_indices, user_adj_values, item_adj_indices, item_adj_values,
           emb_user, emb_item,
           u_W0, u_b0, u_W1, u_b1, i_W0, i_b0, i_W1, i_b1):
    spmm = _make_spmm()

    rows = jnp.stack([user_adj_indices[0], item_adj_indices[0]]) \
        .reshape(2, NSUB, CPT, K)
    cols = jnp.stack([user_adj_indices[1], item_adj_indices[1] + N]) \
        .reshape(2, NSUB, CPT, K)
    vals = jnp.stack([user_adj_values, item_adj_values]) \
        .reshape(2, NSUB, CPT, K)

    w0 = jnp.stack([u_W0, i_W0])
    b0 = jnp.stack([u_b0, i_b0]).reshape(2, 1, D)
    w1 = jnp.stack([u_W1, i_W1])
    b1 = jnp.stack([u_b1, i_b1]).reshape(2, 1, D)

    x = jnp.stack([emb_user, emb_item]).reshape(2 * N, D)
    p = spmm(rows, cols, vals, x)
    x = _dense(p, w0, b0).reshape(2 * N, D)
    p = spmm(rows, cols, vals, x)
    x = _dense(p, w1, b1)
    return (x[0], x[1])
